# baseline (speedup 1.0000x reference)
import math

import jax
import jax.numpy as jnp
from jax import lax
from jax.experimental import pallas as pl
from jax.experimental.pallas import tpu as pltpu

B, N, D = 2, 256, 2
G, F, H = 16, 32, 64
CUTOFF = 1.0
P = 5
EPS = 1e-8
TI = 128
NP = N // 2
E2 = TI * NP
TWO_PI = 2.0 * math.pi
INV_SQRT2 = 1.0 / math.sqrt(2.0)

_EA = -(P + 1) * (P + 2) / 2.0
_EB = P * (P + 2) * 1.0
_EC = -P * (P + 1) / 2.0


def _ss(x):
    return jax.nn.silu(x) * (1.0 / 0.6)


def _geom(rd):
    maskf = ((rd <= CUTOFF) & (rd > 0.0)).astype(jnp.float32)
    ds = rd * (1.0 / CUTOFF)
    ds2 = ds * ds
    ds5 = ds2 * ds2 * ds
    env = (1.0 + _EA * ds5 + _EB * ds5 * ds + _EC * ds5 * ds2)
    env = jnp.where(ds < 1.0, env, 0.0)
    inv_norm = 1.0 / jnp.maximum(jnp.sqrt(jnp.maximum(rd, 1e-16)), EPS)
    return maskf * env * inv_norm


def _rel(sx, sy, dx, dy):
    relx = sx - dx
    rely = sy - dy
    relx = relx - TWO_PI * jnp.round(relx * (1.0 / TWO_PI))
    rely = rely - TWO_PI * jnp.round(rely * (1.0 / TWO_PI))
    return relx, rely, relx * relx + rely * rely


def _body(globs_ref, ci_ref, sxe_ref, sxo_ref, sye_ref, syo_ref,
          dxh_ref, dyh_ref,
          fi_ref, fjp_ref,
          Wg1_ref, Wrd2_ref, Wfi_ref, Wfj2_ref,
          Wr1a_ref, Wr1b_ref, Wgate_ref, Wr2a_ref, Wr2b_ref, Wvec_ref,
          scale_ref, out_ref):
    ci = ci_ref[0]
    fi = fi_ref[0]
    fjp = fjp_ref[0]
    globs = globs_ref[0]
    scale = scale_ref[0, 0]

    relx_e, rely_e, rd_e = _rel(sxe_ref[0], sye_ref[0], dxh_ref[0], dyh_ref[0])
    relx_o, rely_o, rd_o = _rel(sxo_ref[0], syo_ref[0], dxh_ref[0], dyh_ref[0])
    geom_e = _geom(rd_e)
    geom_o = _geom(rd_o)

    gvec = jnp.dot(globs, Wg1_ref[...], preferred_element_type=jnp.float32)
    fip = jnp.dot(fi, Wfi_ref[...], preferred_element_type=jnp.float32)
    bi = fip + gvec
    bi2 = jnp.concatenate([bi, bi], axis=1)
    fjp2 = jnp.dot(fjp, Wfj2_ref[...], preferred_element_type=jnp.float32)

    rdp2 = jnp.concatenate([rd_e[:, :, None], rd_o[:, :, None]], axis=2)
    rd_term = lax.dot_general(rdp2, Wrd2_ref[...],
                              (((2,), (0,)), ((), ())),
                              preferred_element_type=jnp.float32)

    x_pre = rd_term + fjp2[None, :, :] + bi2[:, None, :]
    x = jax.nn.silu(_ss(x_pre)).reshape(E2, 2 * H)

    r = _ss(jnp.dot(_ss(jnp.dot(x, Wr1a_ref[...], preferred_element_type=jnp.float32)),
                    Wr1b_ref[...], preferred_element_type=jnp.float32))
    m = (x + r) * INV_SQRT2
    gate_pre = jnp.dot(m, Wgate_ref[...], preferred_element_type=jnp.float32)
    r2 = _ss(jnp.dot(_ss(jnp.dot(m, Wr2a_ref[...], preferred_element_type=jnp.float32)),
                     Wr2b_ref[...], preferred_element_type=jnp.float32))
    v = jax.nn.silu((m + r2) * INV_SQRT2)
    vw = jnp.dot(v, Wvec_ref[...], preferred_element_type=jnp.float32)

    g3 = gate_pre.reshape(TI, NP, 2)
    v3 = vw.reshape(TI, NP, 2)
    coef_e = (jax.nn.sigmoid(g3[:, :, 0:1].reshape(TI, NP))
              * v3[:, :, 0:1].reshape(TI, NP) * geom_e * scale)
    coef_o = (jax.nn.sigmoid(g3[:, :, 1:2].reshape(TI, NP))
              * v3[:, :, 1:2].reshape(TI, NP) * geom_o * scale)
    aggx = jnp.sum(relx_e * coef_e + relx_o * coef_o, axis=1, keepdims=True)
    aggy = jnp.sum(rely_e * coef_e + rely_o * coef_o, axis=1, keepdims=True)
    out_ref[0] = ci + jnp.concatenate([aggx, aggy], axis=1)


def _blockdiag(W):
    Z = jnp.zeros_like(W)
    return jnp.block([[W, Z], [Z, W]])


def kernel(globs, coors, feats, W_e1, W_r1a, W_r1b, W_gate, W_r2a, W_r2b, W_vec, scale):
    cx = coors[:, :, 0]
    cy = coors[:, :, 1]
    sxe = jnp.broadcast_to(cx[:, None, 0::2], (B, N, NP))
    sxo = jnp.broadcast_to(cx[:, None, 1::2], (B, N, NP))
    sye = jnp.broadcast_to(cy[:, None, 0::2], (B, N, NP))
    syo = jnp.broadcast_to(cy[:, None, 1::2], (B, N, NP))
    dxh = jnp.broadcast_to(cx[:, :, None], (B, N, NP))
    dyh = jnp.broadcast_to(cy[:, :, None], (B, N, NP))
    feats_p = feats.reshape(B, NP, 2 * F)

    Wg1 = W_e1[:G]
    wrd = W_e1[G:G + 1]
    z = jnp.zeros_like(wrd)
    Wrd2 = jnp.concatenate(
        [jnp.concatenate([wrd, z], axis=1),
         jnp.concatenate([z, wrd], axis=1)], axis=0)
    Wfi = W_e1[G + 1:G + 1 + F]
    Wfj2 = _blockdiag(W_e1[G + 1 + F:])
    Wr1a2 = _blockdiag(W_r1a)
    Wr1b2 = _blockdiag(W_r1b)
    Wgate2 = _blockdiag(W_gate)
    Wr2a2 = _blockdiag(W_r2a)
    Wr2b2 = _blockdiag(W_r2b)
    Wvec2 = _blockdiag(W_vec)
    scale2 = scale.reshape(1, 1)

    grid = (B, N // TI)
    full = lambda shape: pl.BlockSpec(shape, lambda b, i: (0,) * len(shape))
    srch = pl.BlockSpec((1, TI, NP), lambda b, i: (b, i, 0))
    out = pl.pallas_call(
        _body,
        grid=grid,
        in_specs=[
            pl.BlockSpec((1, 1, G), lambda b, i: (b, 0, 0)),
            pl.BlockSpec((1, TI, D), lambda b, i: (b, i, 0)),
            srch, srch, srch, srch,
            srch, srch,
            pl.BlockSpec((1, TI, F), lambda b, i: (b, i, 0)),
            pl.BlockSpec((1, NP, 2 * F), lambda b, i: (b, 0, 0)),
            full((G, H)), full((2, 2 * H)), full((F, H)), full((2 * F, 2 * H)),
            full((2 * H, 2 * H)), full((2 * H, 2 * H)), full((2 * H, 2)),
            full((2 * H, 2 * H)), full((2 * H, 2 * H)), full((2 * H, 2)),
            full((1, 1)),
        ],
        out_specs=pl.BlockSpec((1, TI, D), lambda b, i: (b, i, 0)),
        out_shape=jax.ShapeDtypeStruct((B, N, D), jnp.float32),
        compiler_params=pltpu.CompilerParams(
            dimension_semantics=("parallel", "parallel"),
        ),
    )(globs.reshape(B, 1, G), coors, sxe, sxo, sye, syo,
      dxh, dyh, feats, feats_p,
      Wg1, Wrd2, Wfi, Wfj2, Wr1a2, Wr1b2, Wgate2, Wr2a2, Wr2b2, Wvec2, scale2)
    return out


# speedup vs baseline: 1.1769x; 1.1769x over previous
import math

import jax
import jax.numpy as jnp
from jax import lax
from jax.experimental import pallas as pl
from jax.experimental.pallas import tpu as pltpu

B, N, D = 2, 256, 2
G, F, H = 16, 32, 64
CUTOFF = 1.0
P = 5
EPS = 1e-8
TI = 64
NP = N // 2
E2 = TI * NP
TWO_PI = 2.0 * math.pi
INV_SQRT2 = 1.0 / math.sqrt(2.0)

_EA = -(P + 1) * (P + 2) / 2.0
_EB = P * (P + 2) * 1.0
_EC = -P * (P + 1) / 2.0


def _silu(x):
    t = 0.5 * x
    return t + t * jnp.tanh(t)


def _sigmoid(x):
    return 0.5 + 0.5 * jnp.tanh(0.5 * x)


def _ss(x):
    return _silu(x) * (1.0 / 0.6)


def _geom(rd):
    maskf = ((rd <= CUTOFF) & (rd > 0.0)).astype(jnp.float32)
    ds = rd * (1.0 / CUTOFF)
    ds2 = ds * ds
    ds5 = ds2 * ds2 * ds
    env = (1.0 + _EA * ds5 + _EB * ds5 * ds + _EC * ds5 * ds2)
    env = jnp.where(ds < 1.0, env, 0.0)
    inv_norm = 1.0 / jnp.maximum(jnp.sqrt(jnp.maximum(rd, 1e-16)), EPS)
    return maskf * env * inv_norm


def _rel(sx, sy, dx, dy):
    relx = sx - dx
    rely = sy - dy
    relx = relx - TWO_PI * jnp.round(relx * (1.0 / TWO_PI))
    rely = rely - TWO_PI * jnp.round(rely * (1.0 / TWO_PI))
    return relx, rely, relx * relx + rely * rely


def _body(globs_ref, ci_ref, sxe_ref, sxo_ref, sye_ref, syo_ref,
          dxh_ref, dyh_ref,
          fi_ref, fjp_ref,
          Wg1_ref, Wrd2_ref, Wfi_ref, Wfj2_ref,
          Wr1a_ref, Wr1b_ref, Wgate_ref, Wr2a_ref, Wr2b_ref, Wvec_ref,
          scale_ref, out_ref):
    ci = ci_ref[0]
    fi = fi_ref[0]
    fjp = fjp_ref[0]
    globs = globs_ref[0]
    scale = scale_ref[0, 0]

    relx_e, rely_e, rd_e = _rel(sxe_ref[0], sye_ref[0], dxh_ref[0], dyh_ref[0])
    relx_o, rely_o, rd_o = _rel(sxo_ref[0], syo_ref[0], dxh_ref[0], dyh_ref[0])
    geom_e = _geom(rd_e)
    geom_o = _geom(rd_o)

    gvec = jnp.dot(globs, Wg1_ref[...], preferred_element_type=jnp.float32)
    fip = jnp.dot(fi, Wfi_ref[...], preferred_element_type=jnp.float32)
    bi = fip + gvec
    bi2 = jnp.concatenate([bi, bi], axis=1)
    fjp2 = jnp.dot(fjp, Wfj2_ref[...], preferred_element_type=jnp.float32)

    rdp2 = jnp.concatenate([rd_e[:, :, None], rd_o[:, :, None]], axis=2)
    rd_term = lax.dot_general(rdp2, Wrd2_ref[...],
                              (((2,), (0,)), ((), ())),
                              preferred_element_type=jnp.float32)

    x_pre = rd_term + fjp2[None, :, :] + bi2[:, None, :]
    x = _silu(_ss(x_pre)).reshape(E2, 2 * H)

    r = _ss(jnp.dot(_silu(jnp.dot(x, Wr1a_ref[...], preferred_element_type=jnp.float32)),
                    Wr1b_ref[...], preferred_element_type=jnp.float32))
    m = (x + r) * INV_SQRT2
    gate_pre = jnp.dot(m, Wgate_ref[...], preferred_element_type=jnp.float32)
    r2 = _ss(jnp.dot(_silu(jnp.dot(m, Wr2a_ref[...], preferred_element_type=jnp.float32)),
                     Wr2b_ref[...], preferred_element_type=jnp.float32))
    v = _silu((m + r2) * INV_SQRT2)
    vw = jnp.dot(v, Wvec_ref[...], preferred_element_type=jnp.float32)

    g3 = gate_pre.reshape(TI, NP, 2)
    v3 = vw.reshape(TI, NP, 2)
    coef_e = (_sigmoid(g3[:, :, 0:1].reshape(TI, NP))
              * v3[:, :, 0:1].reshape(TI, NP) * geom_e * scale)
    coef_o = (_sigmoid(g3[:, :, 1:2].reshape(TI, NP))
              * v3[:, :, 1:2].reshape(TI, NP) * geom_o * scale)
    aggx = jnp.sum(relx_e * coef_e + relx_o * coef_o, axis=1, keepdims=True)
    aggy = jnp.sum(rely_e * coef_e + rely_o * coef_o, axis=1, keepdims=True)
    out_ref[0] = ci + jnp.concatenate([aggx, aggy], axis=1)


def _blockdiag(W):
    Z = jnp.zeros_like(W)
    return jnp.block([[W, Z], [Z, W]])


def kernel(globs, coors, feats, W_e1, W_r1a, W_r1b, W_gate, W_r2a, W_r2b, W_vec, scale):
    cx = coors[:, :, 0]
    cy = coors[:, :, 1]
    sxe = jnp.broadcast_to(cx[:, None, 0::2], (B, N, NP))
    sxo = jnp.broadcast_to(cx[:, None, 1::2], (B, N, NP))
    sye = jnp.broadcast_to(cy[:, None, 0::2], (B, N, NP))
    syo = jnp.broadcast_to(cy[:, None, 1::2], (B, N, NP))
    dxh = jnp.broadcast_to(cx[:, :, None], (B, N, NP))
    dyh = jnp.broadcast_to(cy[:, :, None], (B, N, NP))
    feats_p = feats.reshape(B, NP, 2 * F)

    Wg1 = W_e1[:G]
    wrd = W_e1[G:G + 1]
    z = jnp.zeros_like(wrd)
    Wrd2 = jnp.concatenate(
        [jnp.concatenate([wrd, z], axis=1),
         jnp.concatenate([z, wrd], axis=1)], axis=0)
    Wfi = W_e1[G + 1:G + 1 + F]
    Wfj2 = _blockdiag(W_e1[G + 1 + F:])
    Wr1a2 = _blockdiag(W_r1a)
    Wr1b2 = _blockdiag(W_r1b) * (1.0 / 0.6)
    Wgate2 = _blockdiag(W_gate)
    Wr2a2 = _blockdiag(W_r2a)
    Wr2b2 = _blockdiag(W_r2b) * (1.0 / 0.6)
    Wvec2 = _blockdiag(W_vec)
    scale2 = scale.reshape(1, 1)

    grid = (B, N // TI)
    full = lambda shape: pl.BlockSpec(shape, lambda b, i: (0,) * len(shape))
    srch = pl.BlockSpec((1, TI, NP), lambda b, i: (b, i, 0))
    out = pl.pallas_call(
        _body,
        grid=grid,
        in_specs=[
            pl.BlockSpec((1, 1, G), lambda b, i: (b, 0, 0)),
            pl.BlockSpec((1, TI, D), lambda b, i: (b, i, 0)),
            srch, srch, srch, srch,
            srch, srch,
            pl.BlockSpec((1, TI, F), lambda b, i: (b, i, 0)),
            pl.BlockSpec((1, NP, 2 * F), lambda b, i: (b, 0, 0)),
            full((G, H)), full((2, 2 * H)), full((F, H)), full((2 * F, 2 * H)),
            full((2 * H, 2 * H)), full((2 * H, 2 * H)), full((2 * H, 2)),
            full((2 * H, 2 * H)), full((2 * H, 2 * H)), full((2 * H, 2)),
            full((1, 1)),
        ],
        out_specs=pl.BlockSpec((1, TI, D), lambda b, i: (b, i, 0)),
        out_shape=jax.ShapeDtypeStruct((B, N, D), jnp.float32),
        compiler_params=pltpu.CompilerParams(
            dimension_semantics=("parallel", "parallel"),
        ),
    )(globs.reshape(B, 1, G), coors, sxe, sxo, sye, syo,
      dxh, dyh, feats, feats_p,
      Wg1, Wrd2, Wfi, Wfj2, Wr1a2, Wr1b2, Wgate2, Wr2a2, Wr2b2, Wvec2, scale2)
    return out
